# baseline (speedup 1.0000x reference)
import functools

import jax
import jax.numpy as jnp
from jax import lax
from jax.experimental import pallas as pl
from jax.experimental.pallas import tpu as pltpu
from jax.experimental.pallas import tpu_sc as plsc

_STRIDE = 1000
_D = 64
_B = 16384
_NC = 2
_NS = 16
_NW = _NC * _NS
_L = 16
_SPLIT = 8192
_BPW = _SPLIT // _NW


def _make_sc_gather():
    mesh = plsc.VectorSubcoreMesh(core_axis_name="c", subcore_axis_name="s")

    @functools.partial(
        pl.kernel,
        mesh=mesh,
        out_type=jax.ShapeDtypeStruct((_SPLIT, _D), jnp.float32),
        scratch_types=[
            pltpu.VMEM((_BPW,), jnp.int32),
            pltpu.VMEM((_BPW,), jnp.int32),
            pltpu.VMEM((_BPW, _D), jnp.float32),
            pltpu.SemaphoreType.DMA,
        ],
    )
    def gather(x1_hbm, x2_hbm, w_hbm, out_hbm, x1_v, x2_v, rows_v, sem):
        wid = lax.axis_index("s") * _NC + lax.axis_index("c")
        base = wid * _BPW
        pltpu.sync_copy(x1_hbm.at[pl.ds(base, _BPW)], x1_v)
        pltpu.sync_copy(x2_hbm.at[pl.ds(base, _BPW)], x2_v)
        copies = []
        for m in range(_BPW // _L):
            a = x1_v[pl.ds(m * _L, _L)]
            b = x2_v[pl.ds(m * _L, _L)]
            fused = a * _STRIDE + b
            for l in range(_L):
                s = lax.squeeze(lax.slice(fused, (l,), (l + 1,)), (0,))
                e = m * _L + l
                copies.append(
                    pltpu.async_copy(w_hbm.at[s], rows_v.at[e], sem)
                )
        for cp in copies:
            cp.wait()
        pltpu.sync_copy(rows_v, out_hbm.at[pl.ds(base, _BPW)])

    return gather


_sc_gather = _make_sc_gather()

_N_TC = _B - _SPLIT


def _tc_body(x1_s, x2_s, w_ref, out_ref, sem):
    def fire(i, carry):
        row = x1_s[_SPLIT + i] * _STRIDE + x2_s[_SPLIT + i]
        pltpu.make_async_copy(
            w_ref.at[pl.ds(row, 1)], out_ref.at[pl.ds(i, 1)], sem
        ).start()
        return carry

    lax.fori_loop(0, _N_TC, fire, 0)

    def drain(i, carry):
        pltpu.make_async_copy(
            w_ref.at[pl.ds(0, 1)], out_ref.at[pl.ds(0, 1)], sem
        ).wait()
        return carry

    lax.fori_loop(0, _N_TC, drain, 0)


_tc_gather = pl.pallas_call(
    _tc_body,
    grid_spec=pltpu.PrefetchScalarGridSpec(
        num_scalar_prefetch=2,
        grid=(1,),
        in_specs=[pl.BlockSpec(memory_space=pl.ANY)],
        out_specs=pl.BlockSpec(memory_space=pl.ANY),
        scratch_shapes=[pltpu.SemaphoreType.DMA],
    ),
    out_shape=jax.ShapeDtypeStruct((_N_TC, _D), jnp.float32),
)


@jax.jit
def kernel(x1, x2, weights):
    x1 = x1.astype(jnp.int32)
    x2 = x2.astype(jnp.int32)
    out_lo = _sc_gather(x1, x2, weights)
    out_hi = _tc_gather(x1, x2, weights)
    return jnp.concatenate([out_lo, out_hi], axis=0)


# speedup vs baseline: 1.4582x; 1.4582x over previous
import functools

import jax
import jax.numpy as jnp
from jax import lax
from jax.experimental import pallas as pl
from jax.experimental.pallas import tpu as pltpu
from jax.experimental.pallas import tpu_sc as plsc

_STRIDE = 1000
_D = 64
_B = 16384
_NC = 2
_NS = 16
_NW = _NC * _NS
_BPW = _B // _NW
_L = 16


def _make_gather():
    mesh = plsc.VectorSubcoreMesh(core_axis_name="c", subcore_axis_name="s")

    @functools.partial(
        pl.kernel,
        mesh=mesh,
        out_type=jax.ShapeDtypeStruct((_B, _D), jnp.float32),
        scratch_types=[
            pltpu.VMEM((_BPW,), jnp.int32),
            pltpu.VMEM((_BPW,), jnp.int32),
            pltpu.VMEM((_BPW, _D), jnp.float32),
            pltpu.SemaphoreType.DMA,
        ],
    )
    def gather(x1_hbm, x2_hbm, w_hbm, out_hbm, x1_v, x2_v, rows_v, sem):
        wid = lax.axis_index("s") * _NC + lax.axis_index("c")
        base = wid * _BPW
        pltpu.sync_copy(x1_hbm.at[pl.ds(base, _BPW)], x1_v)
        pltpu.sync_copy(x2_hbm.at[pl.ds(base, _BPW)], x2_v)

        def fire(m, carry):
            a = x1_v[pl.ds(m * _L, _L)]
            b = x2_v[pl.ds(m * _L, _L)]
            fused = a * _STRIDE + b
            for l in range(_L):
                s = lax.squeeze(lax.slice(fused, (l,), (l + 1,)), (0,))
                pltpu.async_copy(w_hbm.at[s], rows_v.at[m * _L + l], sem)
            return carry

        lax.fori_loop(0, _BPW // _L, fire, 0)

        def drain(i, carry):
            pltpu.make_async_copy(w_hbm.at[0], rows_v.at[0], sem).wait()
            return carry

        lax.fori_loop(0, _BPW, drain, 0)
        pltpu.sync_copy(rows_v, out_hbm.at[pl.ds(base, _BPW)])

    return gather


_gather = _make_gather()


@jax.jit
def kernel(x1, x2, weights):
    out = _gather(x1.astype(jnp.int32), x2.astype(jnp.int32), weights)
    return out.reshape(_B, _D)


# speedup vs baseline: 1.4596x; 1.0010x over previous
import functools

import jax
import jax.numpy as jnp
from jax import lax
from jax.experimental import pallas as pl
from jax.experimental.pallas import tpu as pltpu
from jax.experimental.pallas import tpu_sc as plsc

_STRIDE = 1000
_D = 64
_B = 16384
_NC = 2
_NS = 16
_NW = _NC * _NS
_BPW = _B // _NW
_L = 16


def _make_gather():
    mesh = plsc.VectorSubcoreMesh(core_axis_name="c", subcore_axis_name="s")

    @functools.partial(
        pl.kernel,
        mesh=mesh,
        out_type=jax.ShapeDtypeStruct((_B, _D), jnp.float32),
        scratch_types=[
            pltpu.VMEM((_BPW,), jnp.int32),
            pltpu.VMEM((_BPW,), jnp.int32),
            pltpu.VMEM((_BPW, _D), jnp.float32),
            pltpu.SemaphoreType.DMA,
        ],
    )
    def gather(x1_hbm, x2_hbm, w_hbm, out_hbm, x1_v, x2_v, rows_v, sem):
        wid = lax.axis_index("s") * _NC + lax.axis_index("c")
        base = wid * _BPW
        pltpu.sync_copy(x1_hbm.at[pl.ds(base, _BPW)], x1_v)
        pltpu.sync_copy(x2_hbm.at[pl.ds(base, _BPW)], x2_v)

        def fire(m, carry):
            a = x1_v[pl.ds(m * _L, _L)]
            b = x2_v[pl.ds(m * _L, _L)]
            fused = a * _STRIDE + b
            for l in range(_L):
                s = lax.squeeze(lax.slice(fused, (l,), (l + 1,)), (0,))
                pltpu.async_copy(w_hbm.at[s], rows_v.at[m * _L + l], sem)
            return carry

        def wait_group(m, carry):
            for _ in range(_L):
                pltpu.make_async_copy(w_hbm.at[0], rows_v.at[0], sem).wait()
            return carry

        _K = 8
        n_groups = _BPW // _L
        lax.fori_loop(0, _K, fire, 0)

        def step(m, carry):
            fire(m, carry)
            return wait_group(m, carry)

        lax.fori_loop(_K, n_groups, step, 0)
        lax.fori_loop(0, _K, wait_group, 0)
        pltpu.sync_copy(rows_v, out_hbm.at[pl.ds(base, _BPW)])

    return gather


_gather = _make_gather()


@jax.jit
def kernel(x1, x2, weights):
    out = _gather(x1.astype(jnp.int32), x2.astype(jnp.int32), weights)
    return out.reshape(_B, _D)
